# baseline (speedup 1.0000x reference)
import functools

import jax
import jax.numpy as jnp
from jax import lax
from jax.experimental import pallas as pl
from jax.experimental.pallas import tpu as pltpu
from jax.experimental.pallas import tpu_sc as plsc

_B, _L = 4096, 50
_N = _B * _L
_D = 32
_NT = 4
_NC, _NS = 2, 16
_NW = _NC * _NS
_RPW = _N // _NW
_G = 800
_C = 800
_NCHUNK = _RPW // _C


def _sc_body(i0, i1, i2, i3, t0, t1, t2, t3, out,
             x0, x1, x2, x3, r0, r1, r2, r3,
             isem, gs0, gs1, gs2, gs3, ws0, ws1, ws2, ws3):
    wid = lax.axis_index("s") * _NC + lax.axis_index("c")
    base = wid * _RPW
    ins = (i0, i1, i2, i3)
    tabs = (t0, t1, t2, t3)
    idxs = (x0, x1, x2, x3)
    rows = (r0, r1, r2, r3)
    gsems = (gs0, gs1, gs2, gs3)
    wsems = (ws0, ws1, ws2, ws3)

    icopies = [
        pltpu.async_copy(ins[t].at[pl.ds(base, _RPW)], idxs[t], isem)
        for t in range(_NT)
    ]
    for c in icopies:
        c.wait()

    def fire_gather(ci, t):
        pltpu.async_copy(
            tabs[t].at[idxs[t].at[pl.ds(ci * _C, _C)]],
            rows[t],
            gsems[t],
        )

    def drain_gather(t):
        pltpu.make_async_copy(
            tabs[t].at[idxs[t].at[pl.ds(0, _C)]],
            rows[t],
            gsems[t],
        ).wait()

    def fire_write(ci, t):
        pltpu.async_copy(
            rows[t],
            out.at[pl.ds(base + ci * _C, _C), pl.ds(t * _D, _D)],
            wsems[t],
        )

    def drain_write(t):
        pltpu.make_async_copy(
            rows[t],
            out.at[pl.ds(base, _C), pl.ds(t * _D, _D)],
            wsems[t],
        ).wait()

    for t in range(_NT):
        fire_gather(0, t)

    def chunk(ci, carry):
        for t in range(_NT):
            drain_gather(t)
            fire_write(ci, t)

        @pl.when(ci < _NCHUNK - 1)
        def _():
            for t in range(_NT):
                drain_write(t)
                fire_gather(ci + 1, t)

        return carry

    lax.fori_loop(0, _NCHUNK, chunk, 0)
    for t in range(_NT):
        drain_write(t)


@jax.jit
def _run(i0, i1, i2, i3, t0, t1, t2, t3):
    mesh = plsc.VectorSubcoreMesh(core_axis_name="c", subcore_axis_name="s")
    f = pl.kernel(
        _sc_body,
        out_type=jax.ShapeDtypeStruct((_N, _NT * _D), jnp.float32),
        mesh=mesh,
        scratch_types=(
            [pltpu.VMEM((_RPW,), jnp.int32) for _ in range(_NT)]
            + [pltpu.VMEM((_C, _D), jnp.float32) for _ in range(_NT)]
            + [pltpu.SemaphoreType.DMA] * 9
        ),
        compiler_params=pltpu.CompilerParams(use_tc_tiling_on_sc=False),
    )
    return f(i0, i1, i2, i3, t0, t1, t2, t3)


def kernel(input0, input1, input2, input3, table0, table1, table2, table3):
    idx = [
        jnp.transpose(x).reshape(_N)
        for x in (input0, input1, input2, input3)
    ]
    out = _run(idx[0], idx[1], idx[2], idx[3], table0, table1, table2, table3)
    return out.reshape(_L, _B, _NT * _D).transpose(1, 0, 2)
